# Initial kernel scaffold; baseline (speedup 1.0000x reference)
import jax
import jax.numpy as jnp
from jax.experimental import pallas as pl


def kernel(beta, latent_Z, sample_idx, sparse_sample_i, sparse_sample_j):
    raise NotImplementedError("write your pallas kernel here")


# speedup vs baseline: 2.4067x; 2.4067x over previous
import functools

import jax
import jax.numpy as jnp
from jax import lax
from jax.experimental import pallas as pl
from jax.experimental.pallas import tpu as pltpu
from jax.experimental.pallas import tpu_sc as plsc

N = 10000
D = 64
S = 2000
ES = 12800

S_PAD = 2048
NW = 32
S_PER_W = S_PAD // NW
E_PER_W = ES // NW
LANES = 16

BLK = 256
EPS = 1e-6


def _sc_gather_body(z_hbm, beta_hbm, sidx_hbm, ei_hbm, ej_hbm,
                    zs_hbm, zi_hbm, zj_hbm, bs_hbm, bi_hbm, bj_hbm,
                    idx_s, idx_i, idx_j, rows_s, rows_i, rows_j,
                    bv_s, bv_i, bv_j, sem, bsem):
    wid = lax.axis_index("s") * 2 + lax.axis_index("c")

    def gather_rows(idx_hbm, idx_v, rows_v, out_hbm, bv_v, b_out_hbm, per):
        base = wid * per
        pltpu.sync_copy(idx_hbm.at[pl.ds(base, per)], idx_v)
        zcopy = pltpu.async_copy(z_hbm.at[idx_v], rows_v, sem)
        bcopy = pltpu.async_copy(beta_hbm.at[idx_v], bv_v, bsem)
        zcopy.wait()
        pltpu.sync_copy(rows_v, out_hbm.at[pl.ds(base, per)])
        bcopy.wait()
        pltpu.sync_copy(bv_v, b_out_hbm.at[pl.ds(base, per)])

    gather_rows(sidx_hbm, idx_s, rows_s, zs_hbm, bv_s, bs_hbm, S_PER_W)
    gather_rows(ei_hbm, idx_i, rows_i, zi_hbm, bv_i, bi_hbm, E_PER_W)
    gather_rows(ej_hbm, idx_j, rows_j, zj_hbm, bv_j, bj_hbm, E_PER_W)

@functools.partial(jax.jit, static_argnames=())
def _sc_gather(latent_Z, beta2d, sidx_pad, ei, ej):
    mesh = plsc.VectorSubcoreMesh(core_axis_name="c", subcore_axis_name="s")
    f32 = jnp.float32
    out_type = (
        jax.ShapeDtypeStruct((S_PAD, D), f32),
        jax.ShapeDtypeStruct((ES, D), f32),
        jax.ShapeDtypeStruct((ES, D), f32),
        jax.ShapeDtypeStruct((S_PAD, 1), f32),
        jax.ShapeDtypeStruct((ES, 1), f32),
        jax.ShapeDtypeStruct((ES, 1), f32),
    )
    scratch = [
        pltpu.VMEM((S_PER_W,), jnp.int32),
        pltpu.VMEM((E_PER_W,), jnp.int32),
        pltpu.VMEM((E_PER_W,), jnp.int32),
        pltpu.VMEM((S_PER_W, D), f32),
        pltpu.VMEM((E_PER_W, D), f32),
        pltpu.VMEM((E_PER_W, D), f32),
        pltpu.VMEM((S_PER_W, 1), f32),
        pltpu.VMEM((E_PER_W, 1), f32),
        pltpu.VMEM((E_PER_W, 1), f32),
        pltpu.SemaphoreType.DMA,
        pltpu.SemaphoreType.DMA,
    ]
    k = pl.kernel(_sc_gather_body, out_type=out_type, mesh=mesh,
                  scratch_types=scratch,
                  compiler_params=pltpu.CompilerParams(
                      use_tc_tiling_on_sc=False))
    return k(latent_Z, beta2d, sidx_pad, ei, ej)


def _dense_body(zr_ref, zc_ref, br_ref, bc_ref, out_ref):
    i = pl.program_id(0)
    j = pl.program_id(1)

    @pl.when((i == 0) & (j == 0))
    def _():
        out_ref[...] = jnp.zeros((1, 1), jnp.float32)

    zr = zr_ref[...]
    zc = zc_ref[...]
    g = lax.dot_general(zr, zc, (((1,), (1,)), ((), ())),
                        preferred_element_type=jnp.float32)
    nr = jnp.sum(zr * zr, axis=1, keepdims=True)
    sr = jnp.sum(zr, axis=1, keepdims=True)
    ones_row = jnp.ones((1, D), dtype=jnp.float32)
    nc = lax.dot_general(ones_row, zc * zc, (((1,), (1,)), ((), ())),
                         preferred_element_type=jnp.float32)
    sc = lax.dot_general(ones_row, zc, (((1,), (1,)), ((), ())),
                         preferred_element_type=jnp.float32)

    d2 = nr + nc - 2.0 * g + (2.0 * EPS) * (sr - sc) + (D * EPS * EPS)
    d2 = jnp.maximum(d2, 0.0)
    mat = jnp.exp(br_ref[...] + bc_ref[...] - jnp.sqrt(d2))

    ri = i * BLK + lax.broadcasted_iota(jnp.int32, (BLK, BLK), 0)
    ci = j * BLK + lax.broadcasted_iota(jnp.int32, (BLK, BLK), 1)
    mask = (ri < S) & (ci < S) & (ri != ci)
    out_ref[...] += jnp.sum(jnp.where(mask, mat, 0.0),
                            axis=(0, 1), keepdims=True)


def _dense_sum(zs, bs):
    nblk = S_PAD // BLK
    br = bs
    bc = bs.reshape(1, S_PAD)
    return pl.pallas_call(
        _dense_body,
        grid=(nblk, nblk),
        in_specs=[
            pl.BlockSpec((BLK, D), lambda i, j: (i, 0)),
            pl.BlockSpec((BLK, D), lambda i, j: (j, 0)),
            pl.BlockSpec((BLK, 1), lambda i, j: (i, 0)),
            pl.BlockSpec((1, BLK), lambda i, j: (0, j)),
        ],
        out_specs=pl.BlockSpec((1, 1), lambda i, j: (0, 0)),
        out_shape=jax.ShapeDtypeStruct((1, 1), jnp.float32),
    )(zs, zs, br, bc)


def _edges_body(zi_ref, zj_ref, bi_ref, bj_ref, out_ref):
    diff = zi_ref[...] - zj_ref[...] + EPS
    d2 = jnp.sum(diff * diff, axis=1, keepdims=True)
    dist = jnp.sqrt(d2)
    out_ref[...] = jnp.sum(bi_ref[...] + bj_ref[...] - dist,
                           axis=(0, 1), keepdims=True)


def _edges_sum(zi, zj, bi, bj):
    return pl.pallas_call(
        _edges_body,
        out_shape=jax.ShapeDtypeStruct((1, 1), jnp.float32),
    )(zi, zj, bi, bj)


def kernel(beta, latent_Z, sample_idx, sparse_sample_i, sparse_sample_j):
    sidx = sample_idx.astype(jnp.int32)
    ei = sparse_sample_i.astype(jnp.int32)
    ej = sparse_sample_j.astype(jnp.int32)
    sidx_pad = jnp.concatenate(
        [sidx, jnp.zeros((S_PAD - S,), dtype=jnp.int32)])

    zs, zi, zj, bs, bi, bj = _sc_gather(latent_Z, beta.reshape(N, 1),
                                        sidx_pad, ei, ej)

    offdiag = _dense_sum(zs, bs)
    e2 = jnp.exp(jnp.float32(1.0)) ** 2
    z1 = 0.5 * e2 * offdiag
    z2 = _edges_sum(zi, zj, bi, bj)
    return z2 - z1
